# baseline (speedup 1.0000x reference)
import functools

import jax
import jax.numpy as jnp
from jax import lax
from jax.experimental import pallas as pl
from jax.experimental.pallas import tpu as pltpu
from jax.experimental.pallas import tpu_sc as plsc

_CH = 128


def _sc_gather(B, D, NC, NS):
    NW = NC * NS
    b_per_w = B // NW
    ncb = b_per_w // _CH
    mesh = plsc.VectorSubcoreMesh(core_axis_name="c", subcore_axis_name="s")

    @functools.partial(
        pl.kernel,
        out_type=jax.ShapeDtypeStruct((B, D), jnp.float32),
        mesh=mesh,
        compiler_params=pltpu.CompilerParams(use_tc_tiling_on_sc=False),
        scratch_types=[
            pltpu.VMEM((ncb, _CH), jnp.int32),
            pltpu.VMEM((b_per_w, D), jnp.float32),
            pltpu.SemaphoreType.DMA,
        ],
    )
    def gather_kernel(table_hbm, idx_hbm, out_hbm, idx_v, rows_v, sem):
        wid = lax.axis_index("s") * NC + lax.axis_index("c")
        pltpu.sync_copy(idx_hbm.at[wid], idx_v)
        copies = [
            pltpu.async_copy(
                table_hbm.at[idx_v.at[j]],
                rows_v.at[pl.ds(j * _CH, _CH)],
                sem,
            )
            for j in range(ncb)
        ]
        for c in copies:
            c.wait()
        pltpu.sync_copy(rows_v, out_hbm.at[pl.ds(wid * b_per_w, b_per_w)])

    return gather_kernel


def kernel(emb, node_index):
    V, D = emb.shape
    (B,) = node_index.shape
    info = plsc.get_sparse_core_info()
    NC, NS = info.num_cores, info.num_subcores
    idx3 = node_index.astype(jnp.int32).reshape(NC * NS, -1, _CH)
    return _sc_gather(B, D, NC, NS)(emb, idx3)


# speedup vs baseline: 1.5752x; 1.5752x over previous
import functools

import jax
import jax.numpy as jnp
from jax import lax
from jax.experimental import pallas as pl
from jax.experimental.pallas import tpu as pltpu
from jax.experimental.pallas import tpu_sc as plsc

_WINDOW = 16


def _sc_gather(B, D, NC, NS):
    NW = NC * NS
    b_per_w = B // NW
    mesh = plsc.VectorSubcoreMesh(core_axis_name="c", subcore_axis_name="s")

    @functools.partial(
        pl.kernel,
        out_type=jax.ShapeDtypeStruct((B, D), jnp.float32),
        mesh=mesh,
        compiler_params=pltpu.CompilerParams(use_tc_tiling_on_sc=True),
        scratch_types=[
            pltpu.VMEM((b_per_w,), jnp.int32),
            pltpu.VMEM((b_per_w, D), jnp.float32),
            pltpu.SemaphoreType.DMA,
        ],
    )
    def gather_kernel(table_hbm, idx_hbm, out_hbm, idx_v, rows_v, sem):
        wid = lax.axis_index("s") * NC + lax.axis_index("c")
        base = wid * b_per_w
        pltpu.sync_copy(idx_hbm.at[pl.ds(base, b_per_w)], idx_v)

        def wait_one():
            pltpu.make_async_copy(table_hbm.at[0], rows_v.at[0], sem).wait()

        n_started = 0
        for k in range(b_per_w // 16):
            v = idx_v[pl.ds(k * 16, 16)]
            for j in range(16):
                i = v[j]
                pltpu.async_copy(table_hbm.at[i], rows_v.at[k * 16 + j], sem)
                n_started += 1
                if n_started > _WINDOW:
                    wait_one()
        for _ in range(min(_WINDOW, n_started)):
            wait_one()

        pltpu.sync_copy(rows_v, out_hbm.at[pl.ds(base, b_per_w)])

    return gather_kernel


def kernel(emb, node_index):
    V, D = emb.shape
    (B,) = node_index.shape
    info = plsc.get_sparse_core_info()
    NC, NS = info.num_cores, info.num_subcores
    return _sc_gather(B, D, NC, NS)(emb, node_index.astype(jnp.int32))


# speedup vs baseline: 1.6092x; 1.0216x over previous
import functools

import jax
import jax.numpy as jnp
from jax import lax
from jax.experimental import pallas as pl
from jax.experimental.pallas import tpu as pltpu
from jax.experimental.pallas import tpu_sc as plsc

_BATCH = 16


def _sc_gather(B, D, NC, NS):
    NW = NC * NS
    b_per_w = B // NW
    nb = b_per_w // _BATCH
    mesh = plsc.VectorSubcoreMesh(core_axis_name="c", subcore_axis_name="s")

    @functools.partial(
        pl.kernel,
        out_type=jax.ShapeDtypeStruct((B, D), jnp.float32),
        mesh=mesh,
        compiler_params=pltpu.CompilerParams(use_tc_tiling_on_sc=True),
        scratch_types=[
            pltpu.VMEM((b_per_w,), jnp.int32),
            pltpu.VMEM((b_per_w, D), jnp.float32),
            pltpu.SemaphoreType.DMA,
        ],
    )
    def gather_kernel(table_hbm, idx_hbm, out_hbm, idx_v, rows_v, sem):
        wid = lax.axis_index("s") * NC + lax.axis_index("c")
        base = wid * b_per_w
        pltpu.sync_copy(idx_hbm.at[pl.ds(base, b_per_w)], idx_v)

        def wait_batch():
            pltpu.make_async_copy(
                table_hbm.at[pl.ds(0, _BATCH)],
                rows_v.at[pl.ds(0, _BATCH)],
                sem,
            ).wait()

        for g in range(nb):
            v16 = idx_v[pl.ds(g * _BATCH, 16)]
            for j in range(_BATCH):
                pltpu.async_copy(
                    table_hbm.at[v16[j]],
                    rows_v.at[g * _BATCH + j],
                    sem,
                )
            if g > 0:
                wait_batch()
        wait_batch()

        pltpu.sync_copy(rows_v, out_hbm.at[pl.ds(base, b_per_w)])

    return gather_kernel


def kernel(emb, node_index):
    V, D = emb.shape
    (B,) = node_index.shape
    info = plsc.get_sparse_core_info()
    NC, NS = info.num_cores, info.num_subcores
    return _sc_gather(B, D, NC, NS)(emb, node_index.astype(jnp.int32))


# speedup vs baseline: 1.6284x; 1.0119x over previous
import functools

import jax
import jax.numpy as jnp
from jax import lax
from jax.experimental import pallas as pl
from jax.experimental.pallas import tpu as pltpu
from jax.experimental.pallas import tpu_sc as plsc

_BATCH = 32


def _sc_gather(B, D, NC, NS):
    NW = NC * NS
    b_per_w = B // NW
    nb = b_per_w // _BATCH
    mesh = plsc.VectorSubcoreMesh(core_axis_name="c", subcore_axis_name="s")

    @functools.partial(
        pl.kernel,
        out_type=jax.ShapeDtypeStruct((B, D), jnp.float32),
        mesh=mesh,
        compiler_params=pltpu.CompilerParams(use_tc_tiling_on_sc=True),
        scratch_types=[
            pltpu.VMEM((b_per_w,), jnp.int32),
            pltpu.VMEM((b_per_w, D), jnp.float32),
            pltpu.SemaphoreType.DMA,
        ],
    )
    def gather_kernel(table_hbm, idx_hbm, out_hbm, idx_v, rows_v, sem):
        wid = lax.axis_index("s") * NC + lax.axis_index("c")
        base = wid * b_per_w
        pltpu.sync_copy(idx_hbm.at[pl.ds(base, b_per_w)], idx_v)

        def wait_batch():
            pltpu.make_async_copy(
                table_hbm.at[pl.ds(0, _BATCH)],
                rows_v.at[pl.ds(0, _BATCH)],
                sem,
            ).wait()

        for g in range(nb):
            for h in range(_BATCH // 16):
                v16 = idx_v[pl.ds(g * _BATCH + h * 16, 16)]
                for j in range(16):
                    pltpu.async_copy(
                        table_hbm.at[v16[j]],
                        rows_v.at[g * _BATCH + h * 16 + j],
                        sem,
                    )
            if g > 0:
                wait_batch()
        wait_batch()

        pltpu.sync_copy(rows_v, out_hbm.at[pl.ds(base, b_per_w)])

    return gather_kernel


def kernel(emb, node_index):
    V, D = emb.shape
    (B,) = node_index.shape
    info = plsc.get_sparse_core_info()
    NC, NS = info.num_cores, info.num_subcores
    return _sc_gather(B, D, NC, NS)(emb, node_index.astype(jnp.int32))


# speedup vs baseline: 1.6455x; 1.0105x over previous
import functools

import jax
import jax.numpy as jnp
from jax import lax
from jax.experimental import pallas as pl
from jax.experimental.pallas import tpu as pltpu
from jax.experimental.pallas import tpu_sc as plsc

_BATCH = 64


def _sc_gather(B, D, NC, NS):
    NW = NC * NS
    b_per_w = B // NW
    nb = b_per_w // _BATCH
    mesh = plsc.VectorSubcoreMesh(core_axis_name="c", subcore_axis_name="s")

    @functools.partial(
        pl.kernel,
        out_type=jax.ShapeDtypeStruct((B, D), jnp.float32),
        mesh=mesh,
        compiler_params=pltpu.CompilerParams(use_tc_tiling_on_sc=True),
        scratch_types=[
            pltpu.VMEM((b_per_w,), jnp.int32),
            pltpu.VMEM((b_per_w, D), jnp.float32),
            pltpu.SemaphoreType.DMA,
        ],
    )
    def gather_kernel(table_hbm, idx_hbm, out_hbm, idx_v, rows_v, sem):
        wid = lax.axis_index("s") * NC + lax.axis_index("c")
        base = wid * b_per_w
        pltpu.sync_copy(idx_hbm.at[pl.ds(base, b_per_w)], idx_v)

        def wait_batch():
            pltpu.make_async_copy(
                table_hbm.at[pl.ds(0, _BATCH)],
                rows_v.at[pl.ds(0, _BATCH)],
                sem,
            ).wait()

        for g in range(nb):
            for h in range(_BATCH // 16):
                v16 = idx_v[pl.ds(g * _BATCH + h * 16, 16)]
                for j in range(16):
                    pltpu.async_copy(
                        table_hbm.at[v16[j]],
                        rows_v.at[g * _BATCH + h * 16 + j],
                        sem,
                    )
            if g > 0:
                wait_batch()
        wait_batch()

        pltpu.sync_copy(rows_v, out_hbm.at[pl.ds(base, b_per_w)])

    return gather_kernel


def kernel(emb, node_index):
    V, D = emb.shape
    (B,) = node_index.shape
    info = plsc.get_sparse_core_info()
    NC, NS = info.num_cores, info.num_subcores
    return _sc_gather(B, D, NC, NS)(emb, node_index.astype(jnp.int32))
